# baseline (speedup 1.0000x reference)
import jax
import jax.numpy as jnp
import numpy as np
from jax import lax
from jax.experimental import pallas as pl
from jax.experimental.pallas import tpu as pltpu
from jax.experimental.pallas import tpu_sc as plsc

B = 4096
D = 64
T = 5
NC_TOT = 25
CB = 32
NCHUNK = B // CB
NWORK = 32
CPW = NCHUNK // NWORK

W_OFF, S_OFF, A_OFF, P_OFF, N_OFF = 0, 32, 192, 384, 544
TI_ROWS = 3
TO_ROWS = 4
NBLK = TI_ROWS + TO_ROWS
ROWS = NBLK * 128
IDXR = T * NBLK
DOTS = CB * NC_TOT

_J = np.arange(DOTS)
_B = _J // NC_TOT
_C = _J % NC_TOT
_ROWTAB = np.where(
    _C < 5, P_OFF + _B * 5 + _C,
    np.where(_C < 15, N_OFF + _B * 10 + (_C - 5),
             np.where(_C < 20, S_OFF + _B * 5 + (_C - 15),
                      A_OFF + _B * 5 + (_C - 20)))).astype(np.int32)
_ROWTAB = _ROWTAB.reshape(DOTS // 16, 16)
_BTAB = _B.astype(np.int32).reshape(DOTS // 16, 16)


def _sc_body(ti, to, idx2d, rt, bt, out,
             idx_v, rt_v, bt_v, acc, accf, dotbuf,
             sem_idx, semw1, sem_add, semo):
    nc = 2
    wid = lax.axis_index("s") * nc + lax.axis_index("c")
    pltpu.sync_copy(rt, rt_v)
    pltpu.sync_copy(bt, bt_v)

    def issue_idx(c, s):
        g = wid * CPW + c
        return pltpu.async_copy(idx2d.at[pl.ds(g * IDXR, IDXR)],
                                idx_v.at[s], sem_idx)

    def block_copy(s, t, r, sem, add):
        tab = ti if r < TI_ROWS else to
        ln = 96 if r in (TI_ROWS - 1, NBLK - 1) else 128
        return pltpu.async_copy(
            tab.at[idx_v.at[s, t * NBLK + r, pl.ds(0, ln)]],
            acc.at[s, pl.ds(r * 128, ln)], sem, add=add)

    def issue_wave1(s):
        return [block_copy(s, 0, r, semw1.at[r], add=False)
                for r in range(NBLK)]

    def expand(s):
        shift = jnp.full((16,), 16, jnp.int32)
        mask = jnp.full((16,), -65536, jnp.int32)

        def row_body(r, carry):
            for h in range(2):
                v = acc[s, r, pl.ds(h * 32, 32)]
                iv = plsc.bitcast(v, jnp.int32)
                lo = plsc.bitcast(lax.shift_left(iv, shift), jnp.float32)
                hi = plsc.bitcast(lax.bitwise_and(iv, mask), jnp.float32)
                accf[r, pl.ds(h * 32, 16)] = lo
                accf[r, pl.ds(h * 32 + 16, 16)] = hi
            return carry

        lax.fori_loop(0, ROWS, row_body, 0)

    def dots(s):
        def grp_body(grp, carry2):
            row = rt_v[grp]
            b = bt_v[grp]
            dot = jnp.zeros((16,), jnp.float32)
            for d in range(D):
                dcol = jnp.full((16,), d, jnp.int32)
                ctx = plsc.load_gather(accf, [row, dcol])
                inp = plsc.load_gather(accf, [b, dcol])
                dot = dot + ctx * inp
            dotbuf[s, pl.ds(grp * 16, 16)] = dot
            return carry2

        lax.fori_loop(0, DOTS // 16, grp_body, 0)

    out_cps = [None] * CPW
    idx_cp = issue_idx(0, 0)
    idx_cp.wait()
    w1 = issue_wave1(0)
    for c in range(CPW):
        s = c & 1
        if c + 1 < CPW:
            idx_cp = issue_idx(c + 1, 1 - s)
        add_cps = []
        for r, cp in enumerate(w1):
            cp.wait()
            for t in range(1, T):
                add_cps.append(block_copy(s, t, r, sem_add, add=True))
        for cp in add_cps:
            cp.wait()
        if c + 1 < CPW:
            idx_cp.wait()
            w1 = issue_wave1(1 - s)
        if c >= 2:
            out_cps[c - 2].wait()
        expand(s)
        dots(s)
        out_cps[c] = pltpu.async_copy(
            dotbuf.at[s], out.at[wid * CPW + c], semo.at[s])
    for c in range(CPW - 2, CPW):
        out_cps[c].wait()


def _tc_body(dots_ref, ms_ref, ma_ref, out_ref):
    x = dots_ref[...]
    eps = jnp.float32(1e-10)
    col = lax.broadcasted_iota(jnp.int32, x.shape, 1)

    def sp(v):
        return jnp.maximum(v, 0.0) + jnp.log1p(jnp.exp(-jnp.abs(v)))

    sp_neg = sp(-(x + eps))
    sp_pos = sp(x - eps)
    ms = ms_ref[...]
    ma = ma_ref[...]
    zero = jnp.float32(0.0)
    p_s = jnp.sum(jnp.where(col < 5, sp_neg, zero))
    n_s = jnp.sum(jnp.where((col >= 5) & (col < 15), sp_pos, zero))
    s_s = jnp.sum(jnp.where((col >= 15) & (col < 20), sp_neg * ms, zero))
    a_s = jnp.sum(jnp.where(col >= 20, sp_pos * ma, zero))
    out_ref[0, 0] = p_s
    out_ref[0, 1] = n_s
    out_ref[0, 2] = s_s
    out_ref[0, 3] = a_s


def _pack_indices(w_ix, p_ix, n_ix, s_ix, a_ix):
    zpad = jnp.zeros((NCHUNK, T, 32), jnp.int32)
    wT = w_ix.reshape(NCHUNK, CB, T).transpose(0, 2, 1)
    sT = s_ix.reshape(NCHUNK, CB * 5, T).transpose(0, 2, 1)
    aT = a_ix.reshape(NCHUNK, CB * 5, T).transpose(0, 2, 1)
    pT = p_ix.reshape(NCHUNK, CB * 5, T).transpose(0, 2, 1)
    nT = n_ix.reshape(NCHUNK, CB * 10, T).transpose(0, 2, 1)
    packed = jnp.concatenate([wT, sT, aT, zpad, pT, nT, zpad], axis=2)
    return packed.reshape(NCHUNK * IDXR, 128)


def kernel(w_ix, p_ix, n_ix, s_ix, ms_ix, a_ix, ma_ix, table_i, table_o):
    idx2d = _pack_indices(w_ix, p_ix, n_ix, s_ix, a_ix)

    mesh = plsc.VectorSubcoreMesh(core_axis_name="c", subcore_axis_name="s")
    sc = pl.kernel(
        _sc_body,
        out_type=jax.ShapeDtypeStruct((NCHUNK, DOTS), jnp.float32),
        mesh=mesh,
        scratch_types=[
            pltpu.VMEM((2, IDXR, 128), jnp.int32),
            pltpu.VMEM((DOTS // 16, 16), jnp.int32),
            pltpu.VMEM((DOTS // 16, 16), jnp.int32),
            pltpu.VMEM((2, ROWS, D), jnp.bfloat16),
            pltpu.VMEM((ROWS, D), jnp.float32),
            pltpu.VMEM((2, DOTS), jnp.float32),
            pltpu.SemaphoreType.DMA,
            pltpu.SemaphoreType.DMA((NBLK,)),
            pltpu.SemaphoreType.DMA,
            pltpu.SemaphoreType.DMA((2,)),
        ],
        compiler_params=pltpu.CompilerParams(use_tc_tiling_on_sc=False,
                                             needs_layout_passes=False),
    )
    dots = sc(table_i.astype(jnp.bfloat16), table_o.astype(jnp.bfloat16),
              idx2d,
              jnp.asarray(_ROWTAB), jnp.asarray(_BTAB)).reshape(B, NC_TOT)

    scores = pl.pallas_call(
        _tc_body,
        out_shape=jax.ShapeDtypeStruct((1, 4), jnp.float32),
        out_specs=pl.BlockSpec(memory_space=pltpu.SMEM),
    )(dots, ms_ix, ma_ix)

    p_s = scores[0, 0] / B
    n_s = scores[0, 1] / B
    s_s = scores[0, 2] / B
    a_s = scores[0, 3] / B
    loss = p_s + n_s + s_s + a_s
    return (loss, p_s, n_s, s_s, a_s)


# speedup vs baseline: 1.8449x; 1.8449x over previous
import jax
import jax.numpy as jnp
import numpy as np
from jax import lax
from jax.experimental import pallas as pl
from jax.experimental.pallas import tpu as pltpu
from jax.experimental.pallas import tpu_sc as plsc

B = 4096
D = 64
T = 5
NC_TOT = 25
CB = 32
NCHUNK = B // CB
NWORK = 32
CPW = NCHUNK // NWORK

W_OFF, S_OFF, A_OFF, P_OFF, N_OFF = 0, 32, 192, 384, 544
TI_ROWS = 3
TO_ROWS = 4
NBLK = TI_ROWS + TO_ROWS
ROWS = NBLK * 128
IDXR = T * NBLK
DOTS = CB * NC_TOT

def _sc_body(ti, to, idx2d, out,
             idx_v, acc, dotbuf,
             sem_idx, semw1, sem_add, semo):
    nc = 2
    wid = lax.axis_index("s") * nc + lax.axis_index("c")

    def issue_idx(c, s):
        g = wid * CPW + c
        return pltpu.async_copy(idx2d.at[pl.ds(g * IDXR, IDXR)],
                                idx_v.at[s], sem_idx)

    def block_copy(s, t, r, sem, add):
        tab = ti if r < TI_ROWS else to
        ln = 96 if r in (TI_ROWS - 1, NBLK - 1) else 128
        return pltpu.async_copy(
            tab.at[idx_v.at[s, t * NBLK + r, pl.ds(0, ln)]],
            acc.at[s, pl.ds(r * 128, ln)], sem, add=add)

    def issue_wave1(s):
        return [block_copy(s, 0, r, semw1.at[r], add=False)
                for r in range(NBLK)]

    def dots(s):
        def b_body(b, carry2):
            lane = lax.iota(jnp.int32, 16)
            inp = [acc[s, b, pl.ds(k * 16, 16)] for k in range(4)]
            res = [jnp.zeros((16,), jnp.float32),
                   jnp.zeros((16,), jnp.float32)]

            def one(row, j):
                v = acc[s, row, pl.ds(0, 16)] * inp[0]
                for k in range(1, 4):
                    v = v + acc[s, row, pl.ds(k * 16, 16)] * inp[k]
                tot = jnp.full((16,), jnp.sum(v))
                h, l = divmod(j, 16)
                res[h] = jnp.where(lane == l, tot, res[h])

            for w in range(5):
                one(P_OFF + b * 5 + w, w)
            for w in range(10):
                one(N_OFF + b * 10 + w, 5 + w)
            for w in range(5):
                one(S_OFF + b * 5 + w, 15 + w)
            for w in range(5):
                one(A_OFF + b * 5 + w, 20 + w)
            dotbuf[s, b, pl.ds(0, 16)] = res[0]
            dotbuf[s, b, pl.ds(16, 16)] = res[1]
            return carry2

        lax.fori_loop(0, CB, b_body, 0)

    out_cps = [None] * CPW
    idx_cp = issue_idx(0, 0)
    idx_cp.wait()
    w1 = issue_wave1(0)
    for c in range(CPW):
        s = c & 1
        if c + 1 < CPW:
            idx_cp = issue_idx(c + 1, 1 - s)
        add_cps = []
        for r, cp in enumerate(w1):
            cp.wait()
            for t in range(1, T):
                add_cps.append(block_copy(s, t, r, sem_add, add=True))
        for cp in add_cps:
            cp.wait()
        if c + 1 < CPW:
            idx_cp.wait()
            w1 = issue_wave1(1 - s)
        if c >= 2:
            out_cps[c - 2].wait()
        dots(s)
        out_cps[c] = pltpu.async_copy(
            dotbuf.at[s], out.at[wid * CPW + c], semo.at[s])
    for c in range(CPW - 2, CPW):
        out_cps[c].wait()


def _tc_body(dots_ref, ms_ref, ma_ref, out_ref):
    x = dots_ref[...]
    eps = jnp.float32(1e-10)
    col = lax.broadcasted_iota(jnp.int32, x.shape, 1)

    def sp(v):
        return jnp.maximum(v, 0.0) + jnp.log1p(jnp.exp(-jnp.abs(v)))

    sp_neg = sp(-(x + eps))
    sp_pos = sp(x - eps)
    ms = ms_ref[...]
    ma = ma_ref[...]
    zero = jnp.float32(0.0)
    p_s = jnp.sum(jnp.where(col < 5, sp_neg, zero))
    n_s = jnp.sum(jnp.where((col >= 5) & (col < 15), sp_pos, zero))
    s_s = jnp.sum(jnp.where((col >= 15) & (col < 20), sp_neg * ms, zero))
    a_s = jnp.sum(jnp.where((col >= 20) & (col < 25), sp_pos * ma, zero))
    out_ref[0, 0] = p_s
    out_ref[0, 1] = n_s
    out_ref[0, 2] = s_s
    out_ref[0, 3] = a_s


def _pack_indices(w_ix, p_ix, n_ix, s_ix, a_ix):
    zpad = jnp.zeros((NCHUNK, T, 32), jnp.int32)
    wT = w_ix.reshape(NCHUNK, CB, T).transpose(0, 2, 1)
    sT = s_ix.reshape(NCHUNK, CB * 5, T).transpose(0, 2, 1)
    aT = a_ix.reshape(NCHUNK, CB * 5, T).transpose(0, 2, 1)
    pT = p_ix.reshape(NCHUNK, CB * 5, T).transpose(0, 2, 1)
    nT = n_ix.reshape(NCHUNK, CB * 10, T).transpose(0, 2, 1)
    packed = jnp.concatenate([wT, sT, aT, zpad, pT, nT, zpad], axis=2)
    return packed.reshape(NCHUNK * IDXR, 128)


def kernel(w_ix, p_ix, n_ix, s_ix, ms_ix, a_ix, ma_ix, table_i, table_o):
    idx2d = _pack_indices(w_ix, p_ix, n_ix, s_ix, a_ix)

    mesh = plsc.VectorSubcoreMesh(core_axis_name="c", subcore_axis_name="s")
    sc = pl.kernel(
        _sc_body,
        out_type=jax.ShapeDtypeStruct((NCHUNK, CB, 32), jnp.float32),
        mesh=mesh,
        scratch_types=[
            pltpu.VMEM((2, IDXR, 128), jnp.int32),
            pltpu.VMEM((2, ROWS, D), jnp.float32),
            pltpu.VMEM((2, CB, 32), jnp.float32),
            pltpu.SemaphoreType.DMA,
            pltpu.SemaphoreType.DMA((NBLK,)),
            pltpu.SemaphoreType.DMA,
            pltpu.SemaphoreType.DMA((2,)),
        ],
        compiler_params=pltpu.CompilerParams(use_tc_tiling_on_sc=False,
                                             needs_layout_passes=False),
    )
    dots = sc(table_i, table_o, idx2d).reshape(B, 32)

    scores = pl.pallas_call(
        _tc_body,
        out_shape=jax.ShapeDtypeStruct((1, 4), jnp.float32),
        out_specs=pl.BlockSpec(memory_space=pltpu.SMEM),
    )(dots, ms_ix, ma_ix)

    p_s = scores[0, 0] / B
    n_s = scores[0, 1] / B
    s_s = scores[0, 2] / B
    a_s = scores[0, 3] / B
    loss = p_s + n_s + s_s + a_s
    return (loss, p_s, n_s, s_s, a_s)


# speedup vs baseline: 2.3041x; 1.2489x over previous
import jax
import jax.numpy as jnp
import numpy as np
from jax import lax
from jax.experimental import pallas as pl
from jax.experimental.pallas import tpu as pltpu
from jax.experimental.pallas import tpu_sc as plsc

B = 4096
D = 64
T = 5
NC_TOT = 25
CB = 32
NCHUNK = B // CB
NWORK = 32
CPW = NCHUNK // NWORK

W_OFF, S_OFF, A_OFF, P_OFF, N_OFF = 0, 32, 192, 384, 544
TI_ROWS = 3
TO_ROWS = 4
NBLK = TI_ROWS + TO_ROWS
ROWS = NBLK * 128
IDXR = T * NBLK
DOTS = CB * NC_TOT

def _sc_body(ti, to, idx2d, out,
             idx_v, acc, dotbuf,
             sem_idx, semw1, sem_add, semo):
    nc = 2
    wid = lax.axis_index("s") * nc + lax.axis_index("c")

    def issue_idx(c, s):
        g = wid * CPW + c
        return pltpu.async_copy(idx2d.at[pl.ds(g * IDXR, IDXR)],
                                idx_v.at[s], sem_idx)

    def block_copy(s, t, r, sem, add):
        tab = ti if r < TI_ROWS else to
        ln = 96 if r in (TI_ROWS - 1, NBLK - 1) else 128
        return pltpu.async_copy(
            tab.at[idx_v.at[s, t * NBLK + r, pl.ds(0, ln)]],
            acc.at[s, pl.ds(r * 128, ln)], sem, add=add)

    def issue_wave1(s):
        return [block_copy(s, 0, r, semw1.at[r], add=False)
                for r in range(NBLK)]

    def dots(s):
        def b_body(b, carry2):
            lane = lax.iota(jnp.int32, 16)
            inp = [acc[s, b, pl.ds(k * 16, 16)] for k in range(4)]
            res = [jnp.zeros((16,), jnp.float32),
                   jnp.zeros((16,), jnp.float32)]

            def one(row, j):
                v = acc[s, row, pl.ds(0, 16)] * inp[0]
                for k in range(1, 4):
                    v = v + acc[s, row, pl.ds(k * 16, 16)] * inp[k]
                tot = jnp.full((16,), jnp.sum(v))
                h, l = divmod(j, 16)
                res[h] = jnp.where(lane == l, tot, res[h])

            for w in range(5):
                one(P_OFF + w * CB + b, w)
            for w in range(10):
                one(N_OFF + w * CB + b, 5 + w)
            for w in range(5):
                one(S_OFF + w * CB + b, 15 + w)
            for w in range(5):
                one(A_OFF + w * CB + b, 20 + w)
            dotbuf[s, b, pl.ds(0, 16)] = res[0]
            dotbuf[s, b, pl.ds(16, 16)] = res[1]
            return carry2

        lax.fori_loop(0, CB, b_body, 0)

    out_cps = [None] * CPW
    idx_cp = issue_idx(0, 0)
    idx_cp.wait()
    w1 = issue_wave1(0)
    for c in range(CPW):
        s = c & 1
        if c + 1 < CPW:
            idx_cp = issue_idx(c + 1, 1 - s)
        add_cps = []
        for r, cp in enumerate(w1):
            cp.wait()
            for t in range(1, T):
                add_cps.append(block_copy(s, t, r, sem_add, add=True))
        for cp in add_cps:
            cp.wait()
        if c + 1 < CPW:
            idx_cp.wait()
            w1 = issue_wave1(1 - s)
        if c >= 2:
            out_cps[c - 2].wait()
        dots(s)
        out_cps[c] = pltpu.async_copy(
            dotbuf.at[s], out.at[wid * CPW + c], semo.at[s])
    for c in range(CPW - 2, CPW):
        out_cps[c].wait()


def _tc_body(dots_ref, ms_ref, ma_ref, out_ref):
    x = dots_ref[...]
    eps = jnp.float32(1e-10)
    col = lax.broadcasted_iota(jnp.int32, x.shape, 1)

    def sp(v):
        return jnp.maximum(v, 0.0) + jnp.log1p(jnp.exp(-jnp.abs(v)))

    sp_neg = sp(-(x + eps))
    sp_pos = sp(x - eps)
    ms = ms_ref[...]
    ma = ma_ref[...]
    zero = jnp.float32(0.0)
    p_s = jnp.sum(jnp.where(col < 5, sp_neg, zero))
    n_s = jnp.sum(jnp.where((col >= 5) & (col < 15), sp_pos, zero))
    s_s = jnp.sum(jnp.where((col >= 15) & (col < 20), sp_neg * ms, zero))
    a_s = jnp.sum(jnp.where((col >= 20) & (col < 25), sp_pos * ma, zero))
    out_ref[0, 0] = p_s
    out_ref[0, 1] = n_s
    out_ref[0, 2] = s_s
    out_ref[0, 3] = a_s


def _pack_indices(w_ix, p_ix, n_ix, s_ix, a_ix):
    def tr(x, w):
        y = x.transpose(2, 1, 0).reshape(T, w, NCHUNK, CB)
        return y.transpose(2, 0, 1, 3).reshape(NCHUNK, T, w * CB)

    zpad = jnp.zeros((NCHUNK, T, 32), jnp.int32)
    wT = tr(w_ix.reshape(B, 1, T), 1)
    sT = tr(s_ix, 5)
    aT = tr(a_ix, 5)
    pT = tr(p_ix, 5)
    nT = tr(n_ix, 10)
    packed = jnp.concatenate([wT, sT, aT, zpad, pT, nT, zpad], axis=2)
    return packed.reshape(NCHUNK * IDXR, 128)


def kernel(w_ix, p_ix, n_ix, s_ix, ms_ix, a_ix, ma_ix, table_i, table_o):
    idx2d = _pack_indices(w_ix, p_ix, n_ix, s_ix, a_ix)

    mesh = plsc.VectorSubcoreMesh(core_axis_name="c", subcore_axis_name="s")
    sc = pl.kernel(
        _sc_body,
        out_type=jax.ShapeDtypeStruct((NCHUNK, CB, 32), jnp.float32),
        mesh=mesh,
        scratch_types=[
            pltpu.VMEM((2, IDXR, 128), jnp.int32),
            pltpu.VMEM((2, ROWS, D), jnp.float32),
            pltpu.VMEM((2, CB, 32), jnp.float32),
            pltpu.SemaphoreType.DMA,
            pltpu.SemaphoreType.DMA((NBLK,)),
            pltpu.SemaphoreType.DMA,
            pltpu.SemaphoreType.DMA((2,)),
        ],
        compiler_params=pltpu.CompilerParams(use_tc_tiling_on_sc=False,
                                             needs_layout_passes=False),
    )
    dots = sc(table_i, table_o, idx2d).reshape(B, 32)

    scores = pl.pallas_call(
        _tc_body,
        out_shape=jax.ShapeDtypeStruct((1, 4), jnp.float32),
        out_specs=pl.BlockSpec(memory_space=pltpu.SMEM),
    )(dots, ms_ix, ma_ix)

    p_s = scores[0, 0] / B
    n_s = scores[0, 1] / B
    s_s = scores[0, 2] / B
    a_s = scores[0, 3] / B
    loss = p_s + n_s + s_s + a_s
    return (loss, p_s, n_s, s_s, a_s)
